# speedup vs baseline: 2.3005x; 2.3005x over previous
import functools

import jax
import jax.numpy as jnp
from jax import lax
from jax.experimental import pallas as pl
from jax.experimental.pallas import tpu as pltpu
from jax.experimental.pallas import tpu_sc as plsc

D_MODEL = 1024
VOCAB = 32000
BATCH = 4
SEQ = 2048
IMG_LEN = 256

NTOK = BATCH * SEQ
NC, NS = 2, 16
NW = NC * NS
PER_W = NTOK // NW
CHUNK = 64
NCHUNK = PER_W // CHUNK

OUT_ROWS = BATCH * (IMG_LEN + SEQ)
BLK = 256
BPB = (IMG_LEN + SEQ) // BLK
GRID = OUT_ROWS // BLK
EMB_BLOCKS = NTOK // BLK


@functools.lru_cache(maxsize=None)
def _build_gather():
    mesh = plsc.VectorSubcoreMesh(core_axis_name="c", subcore_axis_name="s")

    @functools.partial(
        pl.kernel,
        mesh=mesh,
        out_type=jax.ShapeDtypeStruct((NTOK, D_MODEL), jnp.float32),
        scratch_types=[
            pltpu.VMEM((CHUNK,), jnp.int32),
            pltpu.VMEM((CHUNK, D_MODEL), jnp.float32),
            pltpu.SemaphoreType.DMA,
        ],
    )
    def _gather(ids_hbm, table_hbm, out_hbm, idx_v, rows_v, sem):
        wid = lax.axis_index("s") * NC + lax.axis_index("c")
        base = wid * PER_W
        for c in range(NCHUNK):
            off = base + c * CHUNK
            pltpu.sync_copy(ids_hbm.at[pl.ds(off, CHUNK)], idx_v)
            pltpu.async_copy(table_hbm.at[idx_v], rows_v, sem).wait()
            pltpu.sync_copy(rows_v, out_hbm.at[pl.ds(off, CHUNK)])

    return _gather


def _mm_body(img_ref, emb_ref, w_ref, b_ref, out_ref):
    jb = pl.program_id(0) % BPB

    @pl.when(jb == 0)
    def _():
        out_ref[...] = (
            jnp.dot(img_ref[...], w_ref[...], preferred_element_type=jnp.float32)
            + b_ref[...]
        )

    @pl.when(jb != 0)
    def _():
        out_ref[...] = (
            jnp.dot(emb_ref[...], w_ref[...], preferred_element_type=jnp.float32)
            + b_ref[...]
        )


@functools.lru_cache(maxsize=None)
def _build_matmul():
    return pl.pallas_call(
        _mm_body,
        grid=(GRID,),
        in_specs=[
            pl.BlockSpec((BLK, D_MODEL), lambda j: (j // BPB, 0)),
            pl.BlockSpec(
                (BLK, D_MODEL),
                lambda j: (jnp.clip(j - j // BPB - 1, 0, EMB_BLOCKS - 1), 0),
            ),
            pl.BlockSpec((D_MODEL, D_MODEL), lambda j: (0, 0)),
            pl.BlockSpec((1, D_MODEL), lambda j: (0, 0)),
        ],
        out_specs=pl.BlockSpec((BLK, D_MODEL), lambda j: (j, 0)),
        out_shape=jax.ShapeDtypeStruct((OUT_ROWS, D_MODEL), jnp.float32),
        compiler_params=pltpu.CompilerParams(
            dimension_semantics=("arbitrary",),
        ),
    )


def kernel(input_ids, image_features, table, W, b):
    ids_flat = input_ids.reshape(NTOK)
    emb = _build_gather()(ids_flat, table)
    img2d = image_features.reshape(BATCH * IMG_LEN, D_MODEL)
    out2d = _build_matmul()(img2d, emb, W, b.reshape(1, D_MODEL))
    return out2d.reshape(BATCH, IMG_LEN + SEQ, D_MODEL)
